# Initial kernel scaffold; baseline (speedup 1.0000x reference)
import jax
import jax.numpy as jnp
from jax.experimental import pallas as pl


def kernel(anchors_weak, anchors_strong, neighbors, ct, h):
    raise NotImplementedError("write your pallas kernel here")


# speedup vs baseline: 1.8341x; 1.8341x over previous
import functools

import jax
import jax.numpy as jnp
from jax.experimental import pallas as pl
from jax.experimental.pallas import tpu as pltpu


def _body(ct_ref, h_ref, aw_ref, as_ref, nb_ref, out_ref, s_acc, c_acc,
          *, num_blocks):
    i = pl.program_id(0)

    @pl.when(i == 0)
    def _init():
        s_acc[...] = jnp.zeros_like(s_acc)
        c_acc[...] = jnp.zeros_like(c_acc)

    ct = ct_ref[0, 0]

    aw = aw_ref[...]
    br, c = aw.shape

    m = jnp.max(aw, axis=1, keepdims=True)
    e = jnp.exp(aw - m)
    s = jnp.sum(e, axis=1, keepdims=True)
    wap = e / s
    maxp = jnp.max(wap, axis=1, keepdims=True)
    maskf = (maxp > ct).astype(jnp.float32)

    colid = jax.lax.broadcasted_iota(jnp.int32, (br, c), 1)
    tgt = jnp.min(jnp.where(wap == maxp, colid, c), axis=1, keepdims=True)
    onehot = (colid == tgt).astype(jnp.float32)
    c_acc[...] += jnp.sum(maskf * onehot, axis=0, keepdims=True)

    awn2 = jnp.sum(aw * aw, axis=1, keepdims=True)
    nb = nb_ref[...]
    nm = jnp.max(nb, axis=2, keepdims=True)
    ne = jnp.exp(nb - nm)
    ns = jnp.sum(ne, axis=2, keepdims=True)
    nbn2 = jnp.sum(nb * nb, axis=2, keepdims=True)
    dots = jnp.sum(aw[:, None, :] * nb, axis=2, keepdims=True)
    cos = dots * jax.lax.rsqrt(awn2[:, :, None] * nbn2)
    d2 = 2.0 - 2.0 * cos
    coef = jnp.exp(-d2) / ns
    beta_un = jnp.sum(coef * ne, axis=1)
    beta = beta_un / jnp.sum(beta_un, axis=1, keepdims=True)

    t = wap - beta
    t2 = jnp.sum(t * t, axis=1, keepdims=True)
    alpha = jnp.minimum(jnp.maximum(1.0, 1.0 / jnp.sqrt(t2)), 100.0)
    q_un = jnp.exp(alpha * (aw - m))
    q = q_un / jnp.sum(q_un, axis=1, keepdims=True)

    a2 = as_ref[...]
    sm = jnp.max(a2, axis=1, keepdims=True)
    sse = jnp.sum(jnp.exp(a2 - sm), axis=1, keepdims=True)
    logp = (a2 - sm) - jnp.log(sse)

    s_acc[...] += jnp.sum((maskf * q) * logp, axis=0, keepdims=True)

    @pl.when(i == num_blocks - 1)
    def _finalize():
        counts = c_acc[...]
        n = jnp.sum(counts)
        freq = counts / n
        h = h_ref[0, 0]
        wt = jnp.where(counts > 0, 1.0 / jnp.log(h + freq), 1.0)
        wt = jnp.clip(wt, 1.0, 50.0)
        w_avg = wt / jnp.sum(wt) * jnp.mean(wt)
        out_ref[...] = jnp.reshape(-jnp.sum(w_avg * s_acc[...]) / n, (1, 1))


def kernel(anchors_weak, anchors_strong, neighbors, ct, h):
    b, c = anchors_weak.shape
    nk = neighbors.shape[1]
    br = 256
    num_blocks = b // br
    ct2 = jnp.reshape(ct.astype(jnp.float32), (1, 1))
    h2 = jnp.reshape(h.astype(jnp.float32), (1, 1))
    out = pl.pallas_call(
        functools.partial(_body, num_blocks=num_blocks),
        grid=(num_blocks,),
        in_specs=[
            pl.BlockSpec(memory_space=pltpu.SMEM),
            pl.BlockSpec(memory_space=pltpu.SMEM),
            pl.BlockSpec((br, c), lambda i: (i, 0)),
            pl.BlockSpec((br, c), lambda i: (i, 0)),
            pl.BlockSpec((br, nk, c), lambda i: (i, 0, 0)),
        ],
        out_specs=pl.BlockSpec((1, 1), lambda i: (0, 0)),
        out_shape=jax.ShapeDtypeStruct((1, 1), jnp.float32),
        scratch_shapes=[
            pltpu.VMEM((1, c), jnp.float32),
            pltpu.VMEM((1, c), jnp.float32),
        ],
        compiler_params=pltpu.CompilerParams(
            dimension_semantics=("arbitrary",)),
    )(ct2, h2, anchors_weak, anchors_strong, neighbors)
    return out[0, 0]
